# Initial kernel scaffold; baseline (speedup 1.0000x reference)
import jax
import jax.numpy as jnp
from jax.experimental import pallas as pl


def kernel(x, edge_index_ext, edge_attr_ext, params):
    raise NotImplementedError("write your pallas kernel here")


# speedup vs baseline: 2.9855x; 2.9855x over previous
import functools

import jax
import jax.numpy as jnp
from jax import lax
from jax.experimental import pallas as pl
from jax.experimental.pallas import tpu as pltpu
from jax.experimental.pallas import tpu_sc as plsc

N_NODES = 10000
N_EDGES = 320000

NC = 2
NS = 16
NW = NC * NS
EPW = N_EDGES // NW
CH = 80
NCHUNK = EPW // CH
NB = 5
NG = NCHUNK // NB
NPS = N_NODES // NS

_f32 = jnp.float32

_sc_params = pltpu.CompilerParams(use_tc_tiling_on_sc=False)


@functools.cache
def _sc_mesh():
    return plsc.VectorSubcoreMesh(core_axis_name="c", subcore_axis_name="s")


def _sc_gather(a_s, a_d, src3, dst3):
    @functools.partial(
        pl.kernel,
        out_type=jax.ShapeDtypeStruct((N_EDGES, 64), _f32),
        mesh=_sc_mesh(),
        scratch_types=[
            pltpu.VMEM((NCHUNK, CH), jnp.int32),
            pltpu.VMEM((NCHUNK, CH), jnp.int32),
            pltpu.VMEM((NB, CH, 64), _f32),
            pltpu.VMEM((NB, CH, 64), _f32),
            pltpu.VMEM((NB, CH, 64), _f32),
            pltpu.SemaphoreType.DMA((NB,)),
            pltpu.SemaphoreType.DMA((NB,)),
        ],
        compiler_params=_sc_params,
    )
    def k(as_hbm, ad_hbm, src_hbm, dst_hbm, out_hbm,
          idxs, idxd, bufs, bufd, outb, sem_g, sem_o):
        wid = lax.axis_index("s") * NC + lax.axis_index("c")
        base = wid * EPW
        pltpu.sync_copy(src_hbm.at[wid], idxs)
        pltpu.sync_copy(dst_hbm.at[wid], idxd)

        for b in range(NB):
            pltpu.async_copy(as_hbm.at[idxs.at[b]], bufs.at[b], sem_g.at[b])
            pltpu.async_copy(ad_hbm.at[idxd.at[b]], bufd.at[b], sem_g.at[b])

        @pl.loop(0, NG)
        def _grp(g):
            for b in range(NB):
                j = g * NB + b
                pltpu.make_async_copy(as_hbm.at[pl.ds(0, CH)], bufs.at[b],
                                      sem_g.at[b]).wait()
                pltpu.make_async_copy(ad_hbm.at[pl.ds(0, CH)], bufd.at[b],
                                      sem_g.at[b]).wait()

                @pl.when(g > 0)
                def _():
                    pltpu.make_async_copy(outb.at[b],
                                          out_hbm.at[pl.ds(0, CH)],
                                          sem_o.at[b]).wait()

                @pl.loop(0, CH)
                def _row(r):
                    for c in range(4):
                        s = pl.ds(c * 16, 16)
                        outb[b, r, s] = bufs[b, r, s] + bufd[b, r, s]

                pltpu.async_copy(outb.at[b], out_hbm.at[pl.ds(base + j * CH, CH)],
                                 sem_o.at[b])

                @pl.when(g < NG - 1)
                def _():
                    jn = j + NB
                    pltpu.async_copy(as_hbm.at[idxs.at[jn]], bufs.at[b],
                                     sem_g.at[b])
                    pltpu.async_copy(ad_hbm.at[idxd.at[jn]], bufd.at[b],
                                     sem_g.at[b])

        for b in range(NB):
            pltpu.make_async_copy(outb.at[b], out_hbm.at[pl.ds(0, CH)],
                                  sem_o.at[b]).wait()

    return k(a_s, a_d, src3, dst3)


def _sc_scatter(rows, dst3, d):
    @functools.partial(
        pl.kernel,
        out_type=jax.ShapeDtypeStruct((NC, N_NODES, d), _f32),
        mesh=_sc_mesh(),
        scratch_types=[
            pltpu.VMEM((NCHUNK, CH), jnp.int32),
            pltpu.VMEM((NB, CH, d), _f32),
            pltpu.VMEM((NPS, d), _f32),
            pltpu.VMEM_SHARED((N_NODES, d), _f32),
            pltpu.SemaphoreType.DMA((NB,)),
            pltpu.SemaphoreType.DMA((NB,)),
        ],
        compiler_params=_sc_params,
    )
    def k(rows_hbm, dst_hbm, out_hbm, idxd, rowv, stage, acc, sem_r, sem_s):
        cid = lax.axis_index("c")
        sid = lax.axis_index("s")
        wid = sid * NC + cid
        base = wid * EPW

        @pl.loop(0, NPS)
        def _z(r):
            for c in range(d // 16):
                stage[r, pl.ds(c * 16, 16)] = jnp.zeros((16,), _f32)

        pltpu.sync_copy(stage, acc.at[pl.ds(sid * NPS, NPS)])
        pltpu.sync_copy(dst_hbm.at[wid], idxd)
        plsc.subcore_barrier()

        for b in range(NB):
            pltpu.async_copy(rows_hbm.at[pl.ds(base + b * CH, CH)], rowv.at[b],
                             sem_r.at[b])

        @pl.loop(0, NG)
        def _grp(g):
            for b in range(NB):
                j = g * NB + b
                pltpu.make_async_copy(rows_hbm.at[pl.ds(0, CH)], rowv.at[b],
                                      sem_r.at[b]).wait()
                pltpu.async_copy(rowv.at[b], acc.at[idxd.at[j]], sem_s.at[b],
                                 add=True)
                pltpu.make_async_copy(rowv.at[b], acc.at[pl.ds(0, CH)],
                                      sem_s.at[b]).wait()

                @pl.when(g < NG - 1)
                def _():
                    pltpu.async_copy(rows_hbm.at[pl.ds(base + (j + NB) * CH, CH)],
                                     rowv.at[b], sem_r.at[b])

        plsc.subcore_barrier()
        pltpu.sync_copy(acc.at[pl.ds(sid * NPS, NPS)], stage)
        pltpu.sync_copy(stage, out_hbm.at[cid, pl.ds(sid * NPS, NPS)])

    return k(rows, dst3)


def _ln(v):
    m = jnp.mean(v, axis=-1, keepdims=True)
    c = v - m
    var = jnp.mean(c * c, axis=-1, keepdims=True)
    return c * lax.rsqrt(var + 1e-5)


def _tc_proj(x, ws, wd):
    def body(x_ref, ws_ref, wd_ref, as_ref, ad_ref):
        xv = x_ref[...]
        as_ref[...] = jnp.dot(xv, ws_ref[...], preferred_element_type=_f32)
        ad_ref[...] = jnp.dot(xv, wd_ref[...], preferred_element_type=_f32)

    return pl.pallas_call(
        body,
        out_shape=(
            jax.ShapeDtypeStruct((N_NODES, 64), _f32),
            jax.ShapeDtypeStruct((N_NODES, 64), _f32),
        ),
    )(x, ws, wd)


_BE = 2000


def _tc_edge(g, e, w0e, b0, w1, b1, mode):
    de = e.shape[1]
    do = w1.shape[1]

    def body(g_ref, e_ref, w0e_ref, b0_ref, w1_ref, b1_ref, *outs):
        ev = e_ref[...]
        pre = g_ref[...] + jnp.dot(ev, w0e_ref[...], preferred_element_type=_f32) \
            + b0_ref[...]
        h = jnp.maximum(pre, 0.0)
        en = jnp.dot(h, w1_ref[...], preferred_element_type=_f32) + b1_ref[...]
        outs[0][...] = en
        if mode == 0:
            outs[1][...] = _ln(en)
        elif mode == 1:
            outs[1][...] = ev + _ln(en)

    full = lambda s: pl.BlockSpec(s, lambda i: (0, 0))
    out_shape = [jax.ShapeDtypeStruct((N_EDGES, do), _f32)]
    out_specs = [pl.BlockSpec((_BE, do), lambda i: (i, 0))]
    if mode != 2:
        out_shape.append(jax.ShapeDtypeStruct((N_EDGES, 64), _f32))
        out_specs.append(pl.BlockSpec((_BE, 64), lambda i: (i, 0)))

    return pl.pallas_call(
        body,
        grid=(N_EDGES // _BE,),
        in_specs=[
            pl.BlockSpec((_BE, 64), lambda i: (i, 0)),
            pl.BlockSpec((_BE, de), lambda i: (i, 0)),
            full((de, 64)),
            full((1, 64)),
            full((64, do)),
            full((1, do)),
        ],
        out_specs=out_specs,
        out_shape=out_shape,
    )(g, e, w0e, b0.reshape(1, 64), w1, b1.reshape(1, do))


def _tc_node(x, p, w0x, w0a, b0, w1, b1, mode, wsn=None, wdn=None):
    def body(x_ref, p_ref, w0x_ref, w0a_ref, b0_ref, w1_ref, b1_ref, *rest):
        xv = x_ref[...]
        agg = p_ref[0] + p_ref[1]
        pre = jnp.dot(xv, w0x_ref[...], preferred_element_type=_f32) \
            + jnp.dot(agg, w0a_ref[...], preferred_element_type=_f32) + b0_ref[...]
        h = jnp.maximum(pre, 0.0)
        nn = jnp.dot(h, w1_ref[...], preferred_element_type=_f32) + b1_ref[...]
        if mode == 0:
            xn = _ln(nn)
        elif mode == 1:
            xn = xv + _ln(nn)
        else:
            xn = nn
        if wsn is None:
            rest[-1][...] = xn
        else:
            wsn_ref, wdn_ref, xn_ref, as_ref, ad_ref = rest
            xn_ref[...] = xn
            as_ref[...] = jnp.dot(xn, wsn_ref[...], preferred_element_type=_f32)
            ad_ref[...] = jnp.dot(xn, wdn_ref[...], preferred_element_type=_f32)

    dn = w1.shape[1]
    args = [x, p, w0x, w0a, b0.reshape(1, 64), w1, b1.reshape(1, dn)]
    outs = [jax.ShapeDtypeStruct((N_NODES, dn), _f32)]
    if wsn is not None:
        args += [wsn, wdn]
        outs += [jax.ShapeDtypeStruct((N_NODES, 64), _f32),
                 jax.ShapeDtypeStruct((N_NODES, 64), _f32)]
    return pl.pallas_call(body, out_shape=tuple(outs))(*args)


def _split_edge_w(p, dx):
    w0 = p["edge"]["W0"]
    return w0[:dx], w0[dx:2 * dx], w0[2 * dx:]


def _layer(p, pn, x, e, a_s, a_d, src3, dst3, mode):
    dx = x.shape[1]
    _, _, we = _split_edge_w(p, dx)
    b0, w1, b1 = p["edge"]["b0"], p["edge"]["W1"], p["edge"]["b1"]

    g = _sc_gather(a_s, a_d, src3, dst3)

    if mode == 2:
        w1p = jnp.zeros((64, 16), _f32).at[:, :1].set(w1)
        b1p = jnp.zeros((16,), _f32).at[:1].set(b1)
        en = _tc_edge(g, e, we, b0, w1p, b1p, mode)[0]
        e_next = en[:, :1]
        d_agg = 16
    else:
        en, e_next = _tc_edge(g, e, we, b0, w1, b1, mode)
        d_agg = 64

    partials = _sc_scatter(en, dst3, d_agg)

    w0n = p["node"]["W0"]
    w0nx, w0na = w0n[:dx], w0n[dx:]
    if mode == 2:
        w0na = jnp.zeros((16, 64), _f32).at[:1].set(w0na)
    if pn is not None:
        wsn, wdn, _ = _split_edge_w(pn, 64)
        xn, a_s2, a_d2 = _tc_node(x, partials, w0nx, w0na, p["node"]["b0"],
                                  p["node"]["W1"], p["node"]["b1"], mode,
                                  wsn, wdn)
    else:
        xn = _tc_node(x, partials, w0nx, w0na, p["node"]["b0"],
                      p["node"]["W1"], p["node"]["b1"], mode)[0]
        a_s2 = a_d2 = None
    return xn, e_next, a_s2, a_d2


def kernel(x, edge_index_ext, edge_attr_ext, params):
    src3 = edge_index_ext[0].reshape(NW, NCHUNK, CH)
    dst3 = edge_index_ext[1].reshape(NW, NCHUNK, CH)

    ws0, wd0, _ = _split_edge_w(params[0], 128)
    a_s, a_d = _tc_proj(x, ws0, wd0)

    xf, ef, a_s, a_d = _layer(params[0], params[1], x, edge_attr_ext,
                              a_s, a_d, src3, dst3, mode=0)
    xf, ef, a_s, a_d = _layer(params[1], params[2], xf, ef,
                              a_s, a_d, src3, dst3, mode=1)
    xf, ef, a_s, a_d = _layer(params[2], params[3], xf, ef,
                              a_s, a_d, src3, dst3, mode=1)
    xf, ef, _, _ = _layer(params[3], None, xf, ef,
                          a_s, a_d, src3, dst3, mode=2)
    return (xf, ef)
